# baseline (speedup 1.0000x reference)
import functools

import jax
import jax.numpy as jnp
from jax import lax
from jax.experimental import pallas as pl
from jax.experimental.pallas import tpu as pltpu
from jax.experimental.pallas import tpu_sc as plsc

CHUNK = 32
NBUF = 2

_TAKE_DNUMS = lax.GatherDimensionNumbers(
    offset_dims=(), collapsed_slice_dims=(0,), start_index_map=(0,))


def _lane_take(x, idx):
    return lax.gather(x, idx[:, None], _TAKE_DNUMS, slice_sizes=(1,),
                      mode=lax.GatherScatterMode.PROMISE_IN_BOUNDS)


def _unpack_pair(w):
    lo = plsc.bitcast(lax.shift_left(w, 16), jnp.float32)
    hi = plsc.bitcast(lax.bitwise_and(w, jnp.int32(-65536)), jnp.float32)
    return lo, hi


@functools.lru_cache(maxsize=None)
def _build(V, D, E_pad, per_tile):
    info = plsc.get_sparse_core_info()
    NC, NS, L = info.num_cores, info.num_subcores, info.num_lanes
    n_chunks = per_tile // CHUNK
    n_outer = -(-n_chunks // NBUF)
    d_regs = D // L
    groups = CHUNK // L

    mesh = plsc.VectorSubcoreMesh(core_axis_name="c", subcore_axis_name="s")

    @functools.partial(
        pl.kernel,
        mesh=mesh,
        out_type=jax.ShapeDtypeStruct((E_pad,), jnp.float32),
        compiler_params=pltpu.CompilerParams(needs_layout_passes=False),
        scratch_types=[
            pltpu.VMEM((per_tile,), jnp.int32),
            pltpu.VMEM((per_tile,), jnp.int32),
            pltpu.VMEM((NBUF, CHUNK, D // 2), jnp.int32),
            pltpu.VMEM((NBUF, CHUNK, D // 2), jnp.int32),
            pltpu.VMEM((per_tile,), jnp.float32),
            pltpu.VMEM((CHUNK * 16,), jnp.float32),
        ] + [pltpu.SemaphoreType.DMA] * (2 * NBUF),
    )
    def k(a_hbm, p_hbm, i0_hbm, i1_hbm, out_hbm,
          i0_v, i1_v, a_buf, p_buf, out_v, red_v, *sems):
        sem_a, sem_p = sems[:NBUF], sems[NBUF:]
        wid = lax.axis_index("s") * NC + lax.axis_index("c")
        base = wid * per_tile
        pltpu.sync_copy(i0_hbm.at[pl.ds(base, per_tile)], i0_v)
        pltpu.sync_copy(i1_hbm.at[pl.ds(base, per_tile)], i1_v)

        iota16 = lax.iota(jnp.int32, L)

        def start(g, b):
            off = pl.multiple_of(g * CHUNK, CHUNK)
            pltpu.async_copy(
                a_hbm.at[i0_v.at[pl.ds(off, CHUNK)]], a_buf.at[b], sem_a[b])
            pltpu.async_copy(
                p_hbm.at[i1_v.at[pl.ds(off, CHUNK)]], p_buf.at[b], sem_p[b])

        def wait(b):
            pltpu.make_async_copy(
                a_hbm.at[pl.ds(0, CHUNK)], a_buf.at[b], sem_a[b]).wait()
            pltpu.make_async_copy(
                p_hbm.at[pl.ds(0, CHUNK)], p_buf.at[b], sem_p[b]).wait()

        def compute(g, b):
            off = pl.multiple_of(g * CHUNK, CHUNK)

            @plsc.parallel_loop(0, CHUNK, step=1, unroll=2)
            def _(e):
                acc = None
                for j in range(D // 2 // L):
                    wa = a_buf[b, e, pl.ds(j * L, L)]
                    wp = p_buf[b, e, pl.ds(j * L, L)]
                    alo, ahi = _unpack_pair(wa)
                    plo, phi = _unpack_pair(wp)
                    t = alo * plo + ahi * phi
                    acc = t if acc is None else acc + t
                red_v[pl.ds(pl.multiple_of(e * L, L), L)] = acc

            for grp in range(groups):
                accs = [red_v[pl.ds((grp * L + i) * L, L)] for i in range(L)]
                accs = [accs[int('{:04b}'.format(i)[::-1], 2)]
                        for i in range(L)]
                width = L
                while len(accs) > 1:
                    width //= 2
                    perm = iota16 ^ width
                    merged = []
                    lane_in_lo = (iota16 & width) == 0
                    for i in range(0, len(accs), 2):
                        u = accs[i]
                        v = accs[i + 1]
                        uf = u + _lane_take(u, perm)
                        vf = v + _lane_take(v, perm)
                        merged.append(
                            jnp.where(lane_in_lo, uf, _lane_take(vf, perm)))
                    accs = merged
                out_v[pl.ds(off + grp * L, L)] = accs[0]

        for b in range(NBUF):
            start(b, b)

        def outer(g0, carry):
            for b in range(NBUF):
                g = g0 * NBUF + b
                plsc.subcore_barrier()

                @pl.when(g < n_chunks)
                def _():
                    wait(b)
                    compute(g, b)

                @pl.when(g + NBUF < n_chunks)
                def _():
                    start(g + NBUF, b)
            return carry

        lax.fori_loop(0, n_outer, outer, 0)
        pltpu.sync_copy(out_v, out_hbm.at[pl.ds(base, per_tile)])

    return k


def kernel(x_author, x_paper, edge_label_index):
    V, D = x_author.shape
    E = edge_label_index.shape[1]
    NW = 32
    per_tile = -(-E // (NW * CHUNK)) * CHUNK
    E_pad = per_tile * NW
    idx = edge_label_index.astype(jnp.int32)
    pad = E_pad - E
    idx0 = jnp.concatenate([idx[0], jnp.zeros((pad,), jnp.int32)])
    idx1 = jnp.concatenate([idx[1], jnp.zeros((pad,), jnp.int32)])
    a16 = lax.bitcast_convert_type(
        x_author.astype(jnp.bfloat16).reshape(V, D // 2, 2), jnp.int32)
    p16 = lax.bitcast_convert_type(
        x_paper.astype(jnp.bfloat16).reshape(V, D // 2, 2), jnp.int32)
    out = _build(V, D, E_pad, per_tile)(a16, p16, idx0, idx1)
    return out[:E]


# speedup vs baseline: 1.4816x; 1.4816x over previous
import functools

import jax
import jax.numpy as jnp
from jax import lax
from jax.experimental import pallas as pl
from jax.experimental.pallas import tpu as pltpu
from jax.experimental.pallas import tpu_sc as plsc

CHUNK = 64
NBUF = 2

_TAKE_DNUMS = lax.GatherDimensionNumbers(
    offset_dims=(), collapsed_slice_dims=(0,), start_index_map=(0,))


def _lane_take(x, idx):
    return lax.gather(x, idx[:, None], _TAKE_DNUMS, slice_sizes=(1,),
                      mode=lax.GatherScatterMode.PROMISE_IN_BOUNDS)


@functools.lru_cache(maxsize=None)
def _build(V, D, E_pad, per_tile):
    info = plsc.get_sparse_core_info()
    NC, NS, L = info.num_cores, info.num_subcores, info.num_lanes
    n_chunks = per_tile // CHUNK
    n_outer = -(-n_chunks // NBUF)
    d_regs = D // L
    groups = CHUNK // L

    mesh = plsc.VectorSubcoreMesh(core_axis_name="c", subcore_axis_name="s")

    @functools.partial(
        pl.kernel,
        mesh=mesh,
        out_type=jax.ShapeDtypeStruct((E_pad,), jnp.float32),
        scratch_types=[
            pltpu.VMEM((per_tile,), jnp.int32),
            pltpu.VMEM((per_tile,), jnp.int32),
            pltpu.VMEM((NBUF, CHUNK, D), jnp.float32),
            pltpu.VMEM((NBUF, CHUNK, D), jnp.float32),
            pltpu.VMEM((per_tile,), jnp.float32),
            pltpu.VMEM((CHUNK * 16,), jnp.float32),
        ] + [pltpu.SemaphoreType.DMA] * (2 * NBUF),
    )
    def k(a_hbm, p_hbm, i0_hbm, i1_hbm, out_hbm,
          i0_v, i1_v, a_buf, p_buf, out_v, red_v, *sems):
        sem_a, sem_p = sems[:NBUF], sems[NBUF:]
        wid = lax.axis_index("s") * NC + lax.axis_index("c")
        base = wid * per_tile
        pltpu.sync_copy(i0_hbm.at[pl.ds(base, per_tile)], i0_v)
        pltpu.sync_copy(i1_hbm.at[pl.ds(base, per_tile)], i1_v)

        iota16 = lax.iota(jnp.int32, L)

        def start(g, b):
            off = pl.multiple_of(g * CHUNK, CHUNK)
            pltpu.async_copy(
                a_hbm.at[i0_v.at[pl.ds(off, CHUNK)]], a_buf.at[b], sem_a[b])
            pltpu.async_copy(
                p_hbm.at[i1_v.at[pl.ds(off, CHUNK)]], p_buf.at[b], sem_p[b])

        def wait(b):
            pltpu.make_async_copy(
                a_hbm.at[pl.ds(0, CHUNK)], a_buf.at[b], sem_a[b]).wait()
            pltpu.make_async_copy(
                p_hbm.at[pl.ds(0, CHUNK)], p_buf.at[b], sem_p[b]).wait()

        def compute(g, b):
            off = pl.multiple_of(g * CHUNK, CHUNK)

            @plsc.parallel_loop(0, CHUNK, step=1, unroll=2)
            def _(e):
                acc = a_buf[b, e, pl.ds(0, L)] * p_buf[b, e, pl.ds(0, L)]
                for j in range(1, d_regs):
                    acc = acc + (a_buf[b, e, pl.ds(j * L, L)]
                                 * p_buf[b, e, pl.ds(j * L, L)])
                red_v[pl.ds(pl.multiple_of(e * L, L), L)] = acc

            for grp in range(groups):
                accs = [red_v[pl.ds((grp * L + i) * L, L)] for i in range(L)]
                accs = [accs[int('{:04b}'.format(i)[::-1], 2)]
                        for i in range(L)]
                width = L
                while len(accs) > 1:
                    width //= 2
                    perm = iota16 ^ width
                    merged = []
                    lane_in_lo = (iota16 & width) == 0
                    for i in range(0, len(accs), 2):
                        u = accs[i]
                        v = accs[i + 1]
                        uf = u + _lane_take(u, perm)
                        vf = v + _lane_take(v, perm)
                        merged.append(
                            jnp.where(lane_in_lo, uf, _lane_take(vf, perm)))
                    accs = merged
                out_v[pl.ds(off + grp * L, L)] = accs[0]

        for b in range(NBUF):
            start(b, b)

        def outer(g0, carry):
            for b in range(NBUF):
                g = g0 * NBUF + b
                plsc.subcore_barrier()

                @pl.when(g < n_chunks)
                def _():
                    wait(b)
                    compute(g, b)

                @pl.when(g + NBUF < n_chunks)
                def _():
                    start(g + NBUF, b)
            return carry

        lax.fori_loop(0, n_outer, outer, 0)
        pltpu.sync_copy(out_v, out_hbm.at[pl.ds(base, per_tile)])

    return k


def kernel(x_author, x_paper, edge_label_index):
    V, D = x_author.shape
    E = edge_label_index.shape[1]
    NW = 32
    per_tile = -(-E // (NW * CHUNK)) * CHUNK
    E_pad = per_tile * NW
    idx = edge_label_index.astype(jnp.int32)
    pad = E_pad - E
    idx0 = jnp.concatenate([idx[0], jnp.zeros((pad,), jnp.int32)])
    idx1 = jnp.concatenate([idx[1], jnp.zeros((pad,), jnp.int32)])
    out = _build(V, D, E_pad, per_tile)(x_author, x_paper, idx0, idx1)
    return out[:E]


# speedup vs baseline: 1.5114x; 1.0202x over previous
import functools

import jax
import jax.numpy as jnp
from jax import lax
from jax.experimental import pallas as pl
from jax.experimental.pallas import tpu as pltpu
from jax.experimental.pallas import tpu_sc as plsc

CHUNK = 16
NBUF = 2

_TAKE_DNUMS = lax.GatherDimensionNumbers(
    offset_dims=(), collapsed_slice_dims=(0,), start_index_map=(0,))


def _lane_take(x, idx):
    return lax.gather(x, idx[:, None], _TAKE_DNUMS, slice_sizes=(1,),
                      mode=lax.GatherScatterMode.PROMISE_IN_BOUNDS)


@functools.lru_cache(maxsize=None)
def _build(V, D, E_pad, per_tile):
    info = plsc.get_sparse_core_info()
    NC, NS, L = info.num_cores, info.num_subcores, info.num_lanes
    n_chunks = per_tile // CHUNK
    n_outer = -(-n_chunks // NBUF)
    d_regs = D // L
    groups = CHUNK // L

    mesh = plsc.VectorSubcoreMesh(core_axis_name="c", subcore_axis_name="s")

    @functools.partial(
        pl.kernel,
        mesh=mesh,
        out_type=jax.ShapeDtypeStruct((E_pad,), jnp.float32),
        scratch_types=[
            pltpu.VMEM((per_tile,), jnp.int32),
            pltpu.VMEM((per_tile,), jnp.int32),
            pltpu.VMEM((NBUF, CHUNK, D), jnp.float32),
            pltpu.VMEM((NBUF, CHUNK, D), jnp.float32),
            pltpu.VMEM((per_tile,), jnp.float32),
            pltpu.VMEM((CHUNK * 16,), jnp.float32),
        ] + [pltpu.SemaphoreType.DMA] * (2 * NBUF),
    )
    def k(a_hbm, p_hbm, i0_hbm, i1_hbm, out_hbm,
          i0_v, i1_v, a_buf, p_buf, out_v, red_v, *sems):
        sem_a, sem_p = sems[:NBUF], sems[NBUF:]
        wid = lax.axis_index("s") * NC + lax.axis_index("c")
        base = wid * per_tile
        pltpu.sync_copy(i0_hbm.at[pl.ds(base, per_tile)], i0_v)
        pltpu.sync_copy(i1_hbm.at[pl.ds(base, per_tile)], i1_v)

        iota16 = lax.iota(jnp.int32, L)

        def start(g, b):
            off = pl.multiple_of(g * CHUNK, CHUNK)
            pltpu.async_copy(
                a_hbm.at[i0_v.at[pl.ds(off, CHUNK)]], a_buf.at[b], sem_a[b])
            pltpu.async_copy(
                p_hbm.at[i1_v.at[pl.ds(off, CHUNK)]], p_buf.at[b], sem_p[b])

        def wait(b):
            pltpu.make_async_copy(
                a_hbm.at[pl.ds(0, CHUNK)], a_buf.at[b], sem_a[b]).wait()
            pltpu.make_async_copy(
                p_hbm.at[pl.ds(0, CHUNK)], p_buf.at[b], sem_p[b]).wait()

        def compute(g, b):
            off = pl.multiple_of(g * CHUNK, CHUNK)

            @plsc.parallel_loop(0, CHUNK, step=1, unroll=2)
            def _(e):
                acc = a_buf[b, e, pl.ds(0, L)] * p_buf[b, e, pl.ds(0, L)]
                for j in range(1, d_regs):
                    acc = acc + (a_buf[b, e, pl.ds(j * L, L)]
                                 * p_buf[b, e, pl.ds(j * L, L)])
                red_v[pl.ds(pl.multiple_of(e * L, L), L)] = acc

            for grp in range(groups):
                accs = [red_v[pl.ds((grp * L + i) * L, L)] for i in range(L)]
                accs = [accs[int('{:04b}'.format(i)[::-1], 2)]
                        for i in range(L)]
                width = L
                while len(accs) > 1:
                    width //= 2
                    perm = iota16 ^ width
                    merged = []
                    lane_in_lo = (iota16 & width) == 0
                    for i in range(0, len(accs), 2):
                        u = accs[i]
                        v = accs[i + 1]
                        uf = u + _lane_take(u, perm)
                        vf = v + _lane_take(v, perm)
                        merged.append(
                            jnp.where(lane_in_lo, uf, _lane_take(vf, perm)))
                    accs = merged
                out_v[pl.ds(off + grp * L, L)] = accs[0]

        for b in range(NBUF):
            start(b, b)

        def outer(g0, carry):
            for b in range(NBUF):
                g = g0 * NBUF + b
                plsc.subcore_barrier()

                @pl.when(g < n_chunks)
                def _():
                    wait(b)
                    compute(g, b)

                @pl.when(g + NBUF < n_chunks)
                def _():
                    start(g + NBUF, b)
            return carry

        lax.fori_loop(0, n_outer, outer, 0)
        pltpu.sync_copy(out_v, out_hbm.at[pl.ds(base, per_tile)])

    return k


def kernel(x_author, x_paper, edge_label_index):
    V, D = x_author.shape
    E = edge_label_index.shape[1]
    NW = 32
    per_tile = -(-E // (NW * CHUNK)) * CHUNK
    E_pad = per_tile * NW
    idx = edge_label_index.astype(jnp.int32)
    pad = E_pad - E
    idx0 = jnp.concatenate([idx[0], jnp.zeros((pad,), jnp.int32)])
    idx1 = jnp.concatenate([idx[1], jnp.zeros((pad,), jnp.int32)])
    out = _build(V, D, E_pad, per_tile)(x_author, x_paper, idx0, idx1)
    return out[:E]


# speedup vs baseline: 2.0115x; 1.3309x over previous
import functools

import jax
import jax.numpy as jnp
from jax import lax
from jax.experimental import pallas as pl
from jax.experimental.pallas import tpu as pltpu
from jax.experimental.pallas import tpu_sc as plsc

CHUNK = 32
NBUF = 4

_TAKE_DNUMS = lax.GatherDimensionNumbers(
    offset_dims=(), collapsed_slice_dims=(0,), start_index_map=(0,))


def _lane_take(x, idx):
    return lax.gather(x, idx[:, None], _TAKE_DNUMS, slice_sizes=(1,),
                      mode=lax.GatherScatterMode.PROMISE_IN_BOUNDS)


@functools.lru_cache(maxsize=None)
def _build(V, D, E_pad, per_tile):
    info = plsc.get_sparse_core_info()
    NC, NS, L = info.num_cores, info.num_subcores, info.num_lanes
    n_chunks = per_tile // CHUNK
    n_outer = -(-n_chunks // NBUF)
    d_regs = D // L
    groups = CHUNK // L

    mesh = plsc.VectorSubcoreMesh(core_axis_name="c", subcore_axis_name="s")

    @functools.partial(
        pl.kernel,
        mesh=mesh,
        out_type=jax.ShapeDtypeStruct((E_pad,), jnp.float32),
        scratch_types=[
            pltpu.VMEM((per_tile,), jnp.int32),
            pltpu.VMEM((per_tile,), jnp.int32),
            pltpu.VMEM((NBUF, CHUNK, D), jnp.float32),
            pltpu.VMEM((NBUF, CHUNK, D), jnp.float32),
            pltpu.VMEM((per_tile,), jnp.float32),
            pltpu.VMEM((CHUNK * 16,), jnp.float32),
        ] + [pltpu.SemaphoreType.DMA] * (2 * NBUF),
    )
    def k(a_hbm, p_hbm, i0_hbm, i1_hbm, out_hbm,
          i0_v, i1_v, a_buf, p_buf, out_v, red_v, *sems):
        sem_a, sem_p = sems[:NBUF], sems[NBUF:]
        wid = lax.axis_index("s") * NC + lax.axis_index("c")
        base = wid * per_tile
        pltpu.sync_copy(i0_hbm.at[pl.ds(base, per_tile)], i0_v)
        pltpu.sync_copy(i1_hbm.at[pl.ds(base, per_tile)], i1_v)

        iota16 = lax.iota(jnp.int32, L)

        def start(g, b):
            off = pl.multiple_of(g * CHUNK, CHUNK)
            pltpu.async_copy(
                a_hbm.at[i0_v.at[pl.ds(off, CHUNK)]], a_buf.at[b], sem_a[b])
            pltpu.async_copy(
                p_hbm.at[i1_v.at[pl.ds(off, CHUNK)]], p_buf.at[b], sem_p[b])

        def wait(b):
            pltpu.make_async_copy(
                a_hbm.at[pl.ds(0, CHUNK)], a_buf.at[b], sem_a[b]).wait()
            pltpu.make_async_copy(
                p_hbm.at[pl.ds(0, CHUNK)], p_buf.at[b], sem_p[b]).wait()

        def compute(g, b):
            off = pl.multiple_of(g * CHUNK, CHUNK)

            @plsc.parallel_loop(0, CHUNK, step=1, unroll=2)
            def _(e):
                acc = a_buf[b, e, pl.ds(0, L)] * p_buf[b, e, pl.ds(0, L)]
                for j in range(1, d_regs):
                    acc = acc + (a_buf[b, e, pl.ds(j * L, L)]
                                 * p_buf[b, e, pl.ds(j * L, L)])
                red_v[pl.ds(pl.multiple_of(e * L, L), L)] = acc

            for grp in range(groups):
                accs = [red_v[pl.ds((grp * L + i) * L, L)] for i in range(L)]
                accs = [accs[int('{:04b}'.format(i)[::-1], 2)]
                        for i in range(L)]
                width = L
                while len(accs) > 1:
                    width //= 2
                    perm = iota16 ^ width
                    merged = []
                    lane_in_lo = (iota16 & width) == 0
                    for i in range(0, len(accs), 2):
                        u = accs[i]
                        v = accs[i + 1]
                        uf = u + _lane_take(u, perm)
                        vf = v + _lane_take(v, perm)
                        merged.append(
                            jnp.where(lane_in_lo, uf, _lane_take(vf, perm)))
                    accs = merged
                out_v[pl.ds(off + grp * L, L)] = accs[0]

        for b in range(NBUF):
            start(b, b)

        def outer(g0, carry):
            for b in range(NBUF):
                g = g0 * NBUF + b
                plsc.subcore_barrier()

                @pl.when(g < n_chunks)
                def _():
                    wait(b)
                    compute(g, b)

                @pl.when(g + NBUF < n_chunks)
                def _():
                    start(g + NBUF, b)
            return carry

        lax.fori_loop(0, n_outer, outer, 0)
        pltpu.sync_copy(out_v, out_hbm.at[pl.ds(base, per_tile)])

    return k


def kernel(x_author, x_paper, edge_label_index):
    V, D = x_author.shape
    E = edge_label_index.shape[1]
    NW = 32
    per_tile = -(-E // (NW * CHUNK)) * CHUNK
    E_pad = per_tile * NW
    idx = edge_label_index.astype(jnp.int32)
    pad = E_pad - E
    idx0 = jnp.concatenate([idx[0], jnp.zeros((pad,), jnp.int32)])
    idx1 = jnp.concatenate([idx[1], jnp.zeros((pad,), jnp.int32)])
    out = _build(V, D, E_pad, per_tile)(x_author, x_paper, idx0, idx1)
    return out[:E]
